# baseline (speedup 1.0000x reference)
import functools

import numpy as np
import jax
import jax.numpy as jnp
from jax import lax
from jax.experimental import pallas as pl
from jax.experimental.pallas import tpu as pltpu
from jax.experimental.pallas import tpu_sc as plsc

SEQ = 1024
D = 768
KEEP_P = 0.9
NC, NS, L = 2, 16, 16
NW = NC * NS

TC_N = 512
SC_N = SEQ - TC_N
BPW = SC_N // NW
WAVE = 64
NWAVES = TC_N // WAVE


def _rotl32(x, r):
    return ((x << np.uint32(r)) | (x >> np.uint32(32 - r))).astype(np.uint32)


def _threefry2x32(ks0, ks1, x0, x1):
    ks = [np.uint32(ks0), np.uint32(ks1),
          np.uint32(ks0 ^ ks1 ^ np.uint32(0x1BD11BDA))]
    x0 = (x0 + ks[0]).astype(np.uint32)
    x1 = (x1 + ks[1]).astype(np.uint32)
    rotations = [[13, 15, 26, 6], [17, 29, 16, 24]]
    for i in range(5):
        for r in rotations[i % 2]:
            x0 = (x0 + x1).astype(np.uint32)
            x1 = _rotl32(x1, r)
            x1 = x1 ^ x0
        x0 = (x0 + ks[(i + 1) % 3]).astype(np.uint32)
        x1 = (x1 + ks[(i + 2) % 3] + np.uint32(i + 1)).astype(np.uint32)
    return x0, x1


def _dropout_consts():
    flat = np.arange(SEQ * D, dtype=np.uint64)
    b0, b1 = _threefry2x32(np.uint32(0), np.uint32(42),
                           (flat >> np.uint64(32)).astype(np.uint32),
                           flat.astype(np.uint32))
    bits = b0 ^ b1
    u = ((bits >> np.uint32(9)) | np.uint32(0x3F800000)).view(np.float32)
    u = np.maximum(np.float32(0.0), u - np.float32(1.0))
    keep = (u < np.float32(KEEP_P)).reshape(SEQ, D)
    scale_tc = np.where(keep[:TC_N], np.float32(1.0 / KEEP_P),
                        np.float32(0.0))
    k4 = keep[TC_N:].reshape(SC_N, D // 64, 4, 16)
    shifts = np.uint32(8) * np.arange(4, dtype=np.uint32)[None, None, :, None]
    words = (k4.astype(np.uint32) << shifts).sum(axis=2, dtype=np.uint32)
    return scale_tc, words.reshape(SC_N, D // 4).view(np.int32)


_scale_tc, _mask_words_sc = _dropout_consts()


_mesh = plsc.VectorSubcoreMesh(
    core_axis_name="c", subcore_axis_name="s", num_cores=NC, num_subcores=NS
)


@functools.partial(
    pl.kernel,
    out_type=jax.ShapeDtypeStruct((SC_N, D), jnp.float32),
    mesh=_mesh,
    scratch_types=[
        pltpu.VMEM((BPW,), jnp.int32),
        pltpu.VMEM((BPW, D), jnp.float32),
        pltpu.VMEM((BPW, D), jnp.float32),
        pltpu.VMEM((BPW, D // 4), jnp.int32),
        pltpu.SemaphoreType.DMA,
    ],
)
def _emb_sc(ids_hbm, wte_hbm, wpe_hbm, mask_hbm, out_hbm,
            idx_v, rows_v, pos_v, msk_v, sem):
    wid = lax.axis_index("s") * NC + lax.axis_index("c")
    base = wid * BPW
    pltpu.sync_copy(ids_hbm.at[pl.ds(TC_N + base, BPW)], idx_v)

    g = pltpu.async_copy(wte_hbm.at[idx_v], rows_v, sem)
    p = pltpu.async_copy(wpe_hbm.at[pl.ds(TC_N + base, BPW)], pos_v, sem)
    s = pltpu.async_copy(mask_hbm.at[pl.ds(base, BPW)], msk_v, sem)
    g.wait()
    p.wait()
    s.wait()

    inv_keep = jnp.float32(1.0 / KEEP_P)

    @plsc.parallel_loop(0, BPW)
    def row_body(j):
        @plsc.parallel_loop(0, D // 64, unroll=2)
        def grp_body(g):
            w = msk_v[j, pl.ds(g * L, L)]
            for k in range(4):
                m = w if k == 0 else lax.shift_right_logical(w, 8 * k)
                mf = (m & 1).astype(jnp.float32)
                sl = pl.ds(g * 64 + k * L, L)
                rows_v[j, sl] = ((rows_v[j, sl] + pos_v[j, sl])
                                 * inv_keep) * mf

    pltpu.sync_copy(rows_v, out_hbm.at[pl.ds(base, BPW)])


def _emb_tc_body(ids_smem, wte_hbm, wpe_hbm, scl_hbm, out_v,
                 rows_v, pos_v, scl_v, sem_rows, sem_aux):
    a1 = pltpu.make_async_copy(wpe_hbm.at[pl.ds(0, TC_N)], pos_v, sem_aux)
    a2 = pltpu.make_async_copy(scl_hbm, scl_v, sem_aux)
    a1.start()
    a2.start()

    def issue(w):
        handles = []
        for r in range(WAVE):
            i = w * WAVE + r
            h = pltpu.make_async_copy(wte_hbm.at[pl.ds(ids_smem[i], 1)],
                                      rows_v.at[pl.ds(i, 1)], sem_rows)
            h.start()
            handles.append(h)
        return handles

    inflight = [issue(0), issue(1)]
    a1.wait()
    a2.wait()
    for w in range(NWAVES):
        if w + 2 < NWAVES:
            inflight.append(issue(w + 2))
        for h in inflight.pop(0):
            h.wait()
        rs = pl.ds(w * WAVE, WAVE)
        out_v[rs, :] = (rows_v[rs, :] + pos_v[rs, :]) * scl_v[rs, :]


_emb_tc = pl.pallas_call(
    _emb_tc_body,
    out_shape=jax.ShapeDtypeStruct((TC_N, D), jnp.float32),
    in_specs=[
        pl.BlockSpec(memory_space=pltpu.SMEM),
        pl.BlockSpec(memory_space=pltpu.HBM),
        pl.BlockSpec(memory_space=pltpu.HBM),
        pl.BlockSpec(memory_space=pltpu.HBM),
    ],
    out_specs=pl.BlockSpec(memory_space=pltpu.VMEM),
    scratch_shapes=[
        pltpu.VMEM((TC_N, D), jnp.float32),
        pltpu.VMEM((TC_N, D), jnp.float32),
        pltpu.VMEM((TC_N, D), jnp.float32),
        pltpu.SemaphoreType.DMA,
        pltpu.SemaphoreType.DMA,
    ],
)


def kernel(input_ids, wte, wpe):
    ids = input_ids.astype(jnp.int32)
    mask_sc = jnp.asarray(_mask_words_sc)
    scale_tc = jnp.asarray(_scale_tc)
    sc_out = _emb_sc(ids, wte, wpe, mask_sc)
    tc_out = _emb_tc(ids, wte, wpe, scale_tc)
    return jnp.concatenate([tc_out, sc_out], axis=0)


# speedup vs baseline: 1.0392x; 1.0392x over previous
import functools

import numpy as np
import jax
import jax.numpy as jnp
from jax import lax
from jax.experimental import pallas as pl
from jax.experimental.pallas import tpu as pltpu
from jax.experimental.pallas import tpu_sc as plsc

SEQ = 1024
D = 768
KEEP_P = 0.9
NC, NS, L = 2, 16, 16
NW = NC * NS
BPW = SEQ // NW
SLICES = D // L

def _rotl32(x, r):
    return ((x << np.uint32(r)) | (x >> np.uint32(32 - r))).astype(np.uint32)


def _threefry2x32(ks0, ks1, x0, x1):
    ks = [np.uint32(ks0), np.uint32(ks1),
          np.uint32(ks0 ^ ks1 ^ np.uint32(0x1BD11BDA))]
    x0 = (x0 + ks[0]).astype(np.uint32)
    x1 = (x1 + ks[1]).astype(np.uint32)
    rotations = [[13, 15, 26, 6], [17, 29, 16, 24]]
    for i in range(5):
        for r in rotations[i % 2]:
            x0 = (x0 + x1).astype(np.uint32)
            x1 = _rotl32(x1, r)
            x1 = x1 ^ x0
        x0 = (x0 + ks[(i + 1) % 3]).astype(np.uint32)
        x1 = (x1 + ks[(i + 2) % 3] + np.uint32(i + 1)).astype(np.uint32)
    return x0, x1


def _dropout_mask_words() -> np.ndarray:
    flat = np.arange(SEQ * D, dtype=np.uint64)
    b0, b1 = _threefry2x32(np.uint32(0), np.uint32(42),
                           (flat >> np.uint64(32)).astype(np.uint32),
                           flat.astype(np.uint32))
    bits = b0 ^ b1
    u = ((bits >> np.uint32(9)) | np.uint32(0x3F800000)).view(np.float32)
    u = np.maximum(np.float32(0.0), u - np.float32(1.0))
    keep = (u < np.float32(KEEP_P)).reshape(SEQ, D // 64, 4, 16)
    shifts = np.uint32(8) * np.arange(4, dtype=np.uint32)[None, None, :, None]
    words = (keep.astype(np.uint32) << shifts).sum(axis=2, dtype=np.uint32)
    return words.reshape(SEQ, D // 4).view(np.int32)


_mask_words = _dropout_mask_words()


_mesh = plsc.VectorSubcoreMesh(
    core_axis_name="c", subcore_axis_name="s", num_cores=NC, num_subcores=NS
)


@functools.partial(
    pl.kernel,
    out_type=jax.ShapeDtypeStruct((SEQ, D), jnp.float32),
    mesh=_mesh,
    scratch_types=[
        pltpu.VMEM((BPW,), jnp.int32),
        pltpu.VMEM((BPW, D), jnp.float32),
        pltpu.VMEM((BPW, D), jnp.float32),
        pltpu.VMEM((BPW, D // 4), jnp.int32),
        pltpu.SemaphoreType.DMA,
        pltpu.SemaphoreType.DMA,
        pltpu.SemaphoreType.DMA,
    ],
)
def _emb_sc(ids_hbm, wte_hbm, wpe_hbm, mask_hbm, out_hbm,
            idx_v, rows_v, pos_v, msk_v, sem_a, sem_b, sem_o):
    wid = lax.axis_index("s") * NC + lax.axis_index("c")
    base = wid * BPW
    pltpu.sync_copy(ids_hbm.at[pl.ds(base, BPW)], idx_v)

    g = pltpu.async_copy(wte_hbm.at[idx_v], rows_v, sem_a)
    p = pltpu.async_copy(wpe_hbm.at[pl.ds(base, BPW)], pos_v, sem_a)
    s = pltpu.async_copy(mask_hbm.at[pl.ds(base, BPW)], msk_v, sem_a)
    g.wait()
    p.wait()
    s.wait()

    inv_keep = jnp.float32(1.0 / KEEP_P)

    @plsc.parallel_loop(0, BPW)
    def row_body(j):
        @plsc.parallel_loop(0, D // 64)
        def grp_body(g):
            w = msk_v[j, pl.ds(g * L, L)]
            for k in range(4):
                m = w if k == 0 else lax.shift_right_logical(w, 8 * k)
                mf = (m & 1).astype(jnp.float32)
                sl = pl.ds(g * 64 + k * L, L)
                rows_v[j, sl] = ((rows_v[j, sl] + pos_v[j, sl])
                                 * inv_keep) * mf

    pltpu.sync_copy(rows_v, out_hbm.at[pl.ds(base, BPW)])


def kernel(input_ids, wte, wpe):
    ids = input_ids.astype(jnp.int32)
    mask = jnp.asarray(_mask_words)
    return _emb_sc(ids, wte, wpe, mask)


# speedup vs baseline: 1.0564x; 1.0165x over previous
import functools

import numpy as np
import jax
import jax.numpy as jnp
from jax import lax
from jax.experimental import pallas as pl
from jax.experimental.pallas import tpu as pltpu
from jax.experimental.pallas import tpu_sc as plsc

SEQ = 1024
D = 768
KEEP_P = 0.9
NC, NS, L = 2, 16, 16
NW = NC * NS
BPW = SEQ // NW
SLICES = D // L

def _rotl32(x, r):
    return ((x << np.uint32(r)) | (x >> np.uint32(32 - r))).astype(np.uint32)


def _threefry2x32(ks0, ks1, x0, x1):
    ks = [np.uint32(ks0), np.uint32(ks1),
          np.uint32(ks0 ^ ks1 ^ np.uint32(0x1BD11BDA))]
    x0 = (x0 + ks[0]).astype(np.uint32)
    x1 = (x1 + ks[1]).astype(np.uint32)
    rotations = [[13, 15, 26, 6], [17, 29, 16, 24]]
    for i in range(5):
        for r in rotations[i % 2]:
            x0 = (x0 + x1).astype(np.uint32)
            x1 = _rotl32(x1, r)
            x1 = x1 ^ x0
        x0 = (x0 + ks[(i + 1) % 3]).astype(np.uint32)
        x1 = (x1 + ks[(i + 2) % 3] + np.uint32(i + 1)).astype(np.uint32)
    return x0, x1


def _dropout_mask_words() -> np.ndarray:
    flat = np.arange(SEQ * D, dtype=np.uint64)
    b0, b1 = _threefry2x32(np.uint32(0), np.uint32(42),
                           (flat >> np.uint64(32)).astype(np.uint32),
                           flat.astype(np.uint32))
    bits = b0 ^ b1
    u = ((bits >> np.uint32(9)) | np.uint32(0x3F800000)).view(np.float32)
    u = np.maximum(np.float32(0.0), u - np.float32(1.0))
    keep = (u < np.float32(KEEP_P)).reshape(SEQ, D // 64, 4, 16)
    shifts = np.uint32(8) * np.arange(4, dtype=np.uint32)[None, None, :, None]
    words = (keep.astype(np.uint32) << shifts).sum(axis=2, dtype=np.uint32)
    return words.reshape(SEQ, D // 4).view(np.int32)


_mask_words = _dropout_mask_words()


_mesh = plsc.VectorSubcoreMesh(
    core_axis_name="c", subcore_axis_name="s", num_cores=NC, num_subcores=NS
)


@functools.partial(
    pl.kernel,
    out_type=jax.ShapeDtypeStruct((SEQ, D), jnp.float32),
    mesh=_mesh,
    scratch_types=[
        pltpu.VMEM((BPW,), jnp.int32),
        pltpu.VMEM((BPW, D), jnp.float32),
        pltpu.VMEM((BPW, D), jnp.float32),
        pltpu.VMEM((BPW, D // 4), jnp.int32),
        pltpu.SemaphoreType.DMA,
        pltpu.SemaphoreType.DMA,
        pltpu.SemaphoreType.DMA,
    ],
)
def _emb_sc(ids_hbm, wte_hbm, wpe_hbm, mask_hbm, out_hbm,
            idx_v, rows_v, pos_v, msk_v, sem_a, sem_b, sem_o):
    wid = lax.axis_index("s") * NC + lax.axis_index("c")
    base = wid * BPW
    H = BPW // 2
    pltpu.sync_copy(ids_hbm.at[pl.ds(base, BPW)], idx_v)

    def fetch(lo, sem):
        g = pltpu.async_copy(wte_hbm.at[idx_v.at[pl.ds(lo, H)]],
                             rows_v.at[pl.ds(lo, H)], sem)
        p = pltpu.async_copy(wpe_hbm.at[pl.ds(base + lo, H)],
                             pos_v.at[pl.ds(lo, H)], sem)
        s = pltpu.async_copy(mask_hbm.at[pl.ds(base + lo, H)],
                             msk_v.at[pl.ds(lo, H)], sem)
        return g, p, s

    inv_keep = jnp.float32(1.0 / KEEP_P)

    def compute(lo):
        @plsc.parallel_loop(lo, lo + H)
        def row_body(j):
            @plsc.parallel_loop(0, D // 64, unroll=2)
            def grp_body(g):
                w = msk_v[j, pl.ds(g * L, L)]
                for k in range(4):
                    m = w if k == 0 else lax.shift_right_logical(w, 8 * k)
                    mf = (m & 1).astype(jnp.float32)
                    sl = pl.ds(g * 64 + k * L, L)
                    rows_v[j, sl] = ((rows_v[j, sl] + pos_v[j, sl])
                                     * inv_keep) * mf

    f0 = fetch(0, sem_a)
    f1 = fetch(H, sem_b)
    for h in f0:
        h.wait()
    compute(0)
    st0 = pltpu.async_copy(rows_v.at[pl.ds(0, H)],
                           out_hbm.at[pl.ds(base, H)], sem_o)
    for h in f1:
        h.wait()
    compute(H)
    st1 = pltpu.async_copy(rows_v.at[pl.ds(H, H)],
                           out_hbm.at[pl.ds(base + H, H)], sem_o)
    st0.wait()
    st1.wait()


def kernel(input_ids, wte, wpe):
    ids = input_ids.astype(jnp.int32)
    mask = jnp.asarray(_mask_words)
    return _emb_sc(ids, wte, wpe, mask)


# speedup vs baseline: 1.0574x; 1.0010x over previous
import functools

import numpy as np
import jax
import jax.numpy as jnp
from jax import lax
from jax.experimental import pallas as pl
from jax.experimental.pallas import tpu as pltpu
from jax.experimental.pallas import tpu_sc as plsc

SEQ = 1024
D = 768
KEEP_P = 0.9
NC, NS, L = 2, 16, 16
NW = NC * NS
BPW = SEQ // NW
SLICES = D // L

def _rotl32(x, r):
    return ((x << np.uint32(r)) | (x >> np.uint32(32 - r))).astype(np.uint32)


def _threefry2x32(ks0, ks1, x0, x1):
    ks = [np.uint32(ks0), np.uint32(ks1),
          np.uint32(ks0 ^ ks1 ^ np.uint32(0x1BD11BDA))]
    x0 = (x0 + ks[0]).astype(np.uint32)
    x1 = (x1 + ks[1]).astype(np.uint32)
    rotations = [[13, 15, 26, 6], [17, 29, 16, 24]]
    for i in range(5):
        for r in rotations[i % 2]:
            x0 = (x0 + x1).astype(np.uint32)
            x1 = _rotl32(x1, r)
            x1 = x1 ^ x0
        x0 = (x0 + ks[(i + 1) % 3]).astype(np.uint32)
        x1 = (x1 + ks[(i + 2) % 3] + np.uint32(i + 1)).astype(np.uint32)
    return x0, x1


def _dropout_mask_words() -> np.ndarray:
    flat = np.arange(SEQ * D, dtype=np.uint64)
    b0, b1 = _threefry2x32(np.uint32(0), np.uint32(42),
                           (flat >> np.uint64(32)).astype(np.uint32),
                           flat.astype(np.uint32))
    bits = b0 ^ b1
    u = ((bits >> np.uint32(9)) | np.uint32(0x3F800000)).view(np.float32)
    u = np.maximum(np.float32(0.0), u - np.float32(1.0))
    keep = (u < np.float32(KEEP_P)).reshape(SEQ, D // 64, 4, 16)
    shifts = np.uint32(8) * np.arange(4, dtype=np.uint32)[None, None, :, None]
    words = (keep.astype(np.uint32) << shifts).sum(axis=2, dtype=np.uint32)
    return words.reshape(SEQ, D // 4).view(np.int32)


_mask_words = _dropout_mask_words()


_mesh = plsc.VectorSubcoreMesh(
    core_axis_name="c", subcore_axis_name="s", num_cores=NC, num_subcores=NS
)


@functools.partial(
    pl.kernel,
    out_type=jax.ShapeDtypeStruct((SEQ, D), jnp.float32),
    mesh=_mesh,
    scratch_types=[
        pltpu.VMEM((BPW,), jnp.int32),
        pltpu.VMEM((BPW, D), jnp.float32),
        pltpu.VMEM((BPW, D), jnp.float32),
        pltpu.VMEM((BPW, D // 4), jnp.int32),
        pltpu.SemaphoreType.DMA,
        pltpu.SemaphoreType.DMA,
        pltpu.SemaphoreType.DMA,
    ],
    compiler_params=pltpu.CompilerParams(skip_device_barrier=True),
)
def _emb_sc(ids_hbm, wte_hbm, wpe_hbm, mask_hbm, out_hbm,
            idx_v, rows_v, pos_v, msk_v, sem_a, sem_b, sem_o):
    wid = lax.axis_index("s") * NC + lax.axis_index("c")
    base = wid * BPW
    H = BPW // 2
    pltpu.sync_copy(ids_hbm.at[pl.ds(base, BPW)], idx_v)

    def fetch(lo, sem):
        g = pltpu.async_copy(wte_hbm.at[idx_v.at[pl.ds(lo, H)]],
                             rows_v.at[pl.ds(lo, H)], sem)
        p = pltpu.async_copy(wpe_hbm.at[pl.ds(base + lo, H)],
                             pos_v.at[pl.ds(lo, H)], sem)
        s = pltpu.async_copy(mask_hbm.at[pl.ds(base + lo, H)],
                             msk_v.at[pl.ds(lo, H)], sem)
        return g, p, s

    inv_keep = jnp.float32(1.0 / KEEP_P)

    def compute(lo):
        @plsc.parallel_loop(lo, lo + H)
        def row_body(j):
            @plsc.parallel_loop(0, D // 64, unroll=2)
            def grp_body(g):
                w = msk_v[j, pl.ds(g * L, L)]
                for k in range(4):
                    m = w if k == 0 else lax.shift_right_logical(w, 8 * k)
                    mf = (m & 1).astype(jnp.float32)
                    sl = pl.ds(g * 64 + k * L, L)
                    rows_v[j, sl] = ((rows_v[j, sl] + pos_v[j, sl])
                                     * inv_keep) * mf

    f0 = fetch(0, sem_a)
    f1 = fetch(H, sem_b)
    for h in f0:
        h.wait()
    compute(0)
    st0 = pltpu.async_copy(rows_v.at[pl.ds(0, H)],
                           out_hbm.at[pl.ds(base, H)], sem_o)
    for h in f1:
        h.wait()
    compute(H)
    st1 = pltpu.async_copy(rows_v.at[pl.ds(H, H)],
                           out_hbm.at[pl.ds(base + H, H)], sem_o)
    st0.wait()
    st1.wait()


def kernel(input_ids, wte, wpe):
    ids = input_ids.astype(jnp.int32)
    mask = jnp.asarray(_mask_words)
    return _emb_sc(ids, wte, wpe, mask)
